# baseline (speedup 1.0000x reference)
import functools

import jax
import jax.numpy as jnp
from jax import lax
from jax.experimental import pallas as pl
from jax.experimental.pallas import tpu as pltpu
from jax.experimental.pallas import tpu_sc as plsc

LATENT = 64
XDIM = 16
UDIM = LATENT + XDIM
LANES = 16
WIDTH = 128
IDX_CHUNK = 128
SELCAP = 640
NRING = 2


def kernel(users, items, xij, emb_user, emb_item, emb_item_xij1, emb_item_xij0):
    B = users.shape[0]
    NU = emb_user.shape[0]
    info = plsc.get_sparse_core_info()
    NC, NS = info.num_cores, info.num_subcores
    NW = NC * NS
    assert B % (NW * LANES) == 0
    RPW = B // NW
    NCHUNK = RPW // IDX_CHUNK
    NCOL = (NU + 127) // 128
    CPW = (NCOL + NW - 1) // NW
    BPAD = B + IDX_CHUNK
    NSEL = SELCAP // IDX_CHUNK

    ut = emb_user.T
    proj_i = jnp.eye(3 * LATENT // 2, WIDTH, dtype=jnp.float32)
    icat = jnp.concatenate(
        [jnp.exp(emb_item), emb_item_xij1, emb_item_xij0], axis=1) @ proj_i

    mesh = plsc.VectorSubcoreMesh(core_axis_name="c", subcore_axis_name="s")
    cparams = pltpu.CompilerParams(
        needs_layout_passes=False, use_tc_tiling_on_sc=True)

    @functools.partial(
        pl.kernel,
        out_type=jax.ShapeDtypeStruct((BPAD, WIDTH), jnp.float32),
        mesh=mesh,
        scratch_types=[
            pltpu.VMEM((B,), jnp.int32),
            pltpu.VMEM((NSEL, IDX_CHUNK), jnp.int32),
            pltpu.VMEM((SELCAP,), jnp.int32),
            pltpu.VMEM((NRING * UDIM, 128), jnp.float32),
            pltpu.VMEM((SELCAP, WIDTH), jnp.float32),
            pltpu.SemaphoreType.DMA,
            pltpu.SemaphoreType.DMA,
        ],
        compiler_params=cparams,
    )
    def extract_kernel(users_h, ut_h, out_h, uall, selr, selu, ring, orows,
                       sem_in, sem_out):
        wid = lax.axis_index("s") * NC + lax.axis_index("c")
        lo = wid * CPW
        lanes = lax.broadcasted_iota(jnp.int32, (LANES,), 0)

        pltpu.async_copy(users_h, uall, sem_in).wait()

        def prefill(g, carry):
            idx = g * LANES + lanes
            plsc.store_scatter(selr, [lax.shift_right_logical(idx, 7),
                                      jnp.bitwise_and(idx, 127)],
                               jnp.full((LANES,), B, jnp.int32))
            selu[pl.ds(g * LANES, LANES)] = jnp.full((LANES,), 0x7FFFFFF,
                                                     jnp.int32)
            return carry
        lax.fori_loop(0, SELCAP // LANES, prefill, 0)

        def select(g, off):
            u = uall[pl.ds(g * LANES, LANES)]
            tc = lax.shift_right_logical(u, 7) - lo
            m = (tc >= 0) & (tc < CPW)
            mi = m.astype(jnp.int32)
            pos = off + plsc.cumsum(mi) - 1
            m = m & (pos < SELCAP)
            plsc.store_scatter(selu, [pos], u, mask=m)
            plsc.store_scatter(selr, [lax.shift_right_logical(pos, 7),
                                      jnp.bitwise_and(pos, 127)],
                               g * LANES + lanes, mask=m)
            return off + lax.reduce_sum(mi, axes=(0,))
        nsel = lax.fori_loop(0, B // LANES, select, jnp.int32(0))

        nsg = lax.div(nsel + LANES - 1, LANES)

        def fire(c):
            col = lo + c
            slot = lax.rem(c, NRING)

            @pl.when((col < NCOL) & (c < CPW))
            def _():
                pltpu.async_copy(
                    ut_h.at[:, pl.ds(col * 128, 128)],
                    ring.at[pl.ds(slot * UDIM, UDIM)], sem_in)

        for c in range(NRING - 1):
            fire(c)

        one = jnp.float32(1.0)

        def colbody(c, carry):
            col = lo + c
            slot = lax.rem(c, NRING)

            @pl.when(col < NCOL)
            def _wait():
                pltpu.make_async_copy(
                    ut_h.at[:, pl.ds(col * 128, 128)],
                    ring.at[pl.ds(slot * UDIM, UDIM)], sem_in).wait()

            fire(c + NRING - 1)

            @pl.when(col < NCOL)
            def _process():
                def scan_groups(g, carry2):
                    su = selu[pl.ds(g * LANES, LANES)]
                    m2 = lax.shift_right_logical(su, 7) == col

                    @pl.when(jnp.any(m2))
                    def _extract():
                        def member_loop(state):
                            mm, acc = state
                            lanev = plsc.all_reduce_ffs(mm)
                            uv = plsc.load_gather(selu, [g * LANES + lanev])
                            ucol = jnp.bitwise_and(uv, 127)
                            pos_v = g * LANES + lanev
                            for k in range(UDIM // LANES):
                                vals = plsc.load_gather(
                                    ring, [slot * UDIM + k * LANES + lanes,
                                           ucol])
                                sig = one / (one + jnp.exp(-vals))
                                plsc.store_scatter(
                                    orows, [pos_v, k * LANES + lanes], sig)
                            mm = mm & (lanes != lanev)
                            return (mm, acc)

                        lax.while_loop(lambda s: jnp.any(s[0]), member_loop,
                                       (m2, 0))
                    return carry2

                lax.fori_loop(0, nsg, scan_groups, 0)
            return carry

        lax.fori_loop(0, CPW, colbody, 0)

        sc_copies = []
        for j in range(NSEL):
            sc_copies.append(pltpu.async_copy(
                orows.at[pl.ds(j * IDX_CHUNK, IDX_CHUNK)],
                out_h.at[selr.at[j]], sem_out))
        for cp in sc_copies:
            cp.wait()

    @functools.partial(
        pl.kernel,
        out_type=jax.ShapeDtypeStruct((B,), jnp.float32),
        mesh=mesh,
        scratch_types=[
            pltpu.VMEM((RPW,), jnp.int32),
            pltpu.VMEM((RPW,), jnp.float32),
            pltpu.VMEM((RPW, WIDTH), jnp.float32),
            pltpu.VMEM((2 * IDX_CHUNK, WIDTH), jnp.float32),
            pltpu.VMEM((RPW,), jnp.float32),
            pltpu.SemaphoreType.DMA,
            pltpu.SemaphoreType.DMA,
        ],
        compiler_params=cparams,
    )
    def combine_kernel(items_h, xij_h, us_h, ic_h, out_h,
                       iidx, xv, ubuf, ibuf, outv, sem_a, sem_b):
        wid = lax.axis_index("s") * NC + lax.axis_index("c")
        base = wid * RPW

        bsl = pl.ds(base, RPW)
        c1 = pltpu.async_copy(items_h.at[bsl], iidx, sem_a)
        c2 = pltpu.async_copy(xij_h.at[bsl], xv, sem_a)
        c3 = pltpu.async_copy(us_h.at[bsl, :], ubuf, sem_b)
        c1.wait()
        c2.wait()
        c3.wait()

        lanes = lax.broadcasted_iota(jnp.int32, (LANES,), 0)
        one = jnp.float32(1.0)

        def fire(q, sem):
            sl = pl.ds((q % 2) * IDX_CHUNK, IDX_CHUNK)
            qsl = pl.ds(q * IDX_CHUNK, IDX_CHUNK)
            return (pltpu.async_copy(ic_h.at[iidx.at[qsl]], ibuf.at[sl], sem),)

        def compute(q):
            def group_body(g, carry):
                brows = (q % 2) * IDX_CHUNK + g * LANES + lanes
                off = q * IDX_CHUNK + g * LANES
                urows = off + lanes
                x = xv[pl.ds(off, LANES)]
                denom = jnp.zeros((LANES,), jnp.float32)
                numer = jnp.zeros((LANES,), jnp.float32)
                for d in range(LATENT):
                    dd = jnp.full((LANES,), d, jnp.int32)
                    e = plsc.load_gather(ibuf, [brows, dd])
                    s = plsc.load_gather(ubuf, [urows, dd])
                    denom = denom + e
                    numer = numer + s * e
                for d in range(XDIM):
                    d1 = jnp.full((LANES,), LATENT + d, jnp.int32)
                    d0 = jnp.full((LANES,), LATENT + XDIM + d, jnp.int32)
                    x1 = plsc.load_gather(ibuf, [brows, d1])
                    x0 = plsc.load_gather(ibuf, [brows, d0])
                    e = jnp.exp(x1 * x + x0 * (one - x))
                    s = plsc.load_gather(ubuf, [urows, d1])
                    denom = denom + e
                    numer = numer + s * e
                outv[pl.ds(off, LANES)] = numer / denom
                return carry

            lax.fori_loop(0, IDX_CHUNK // LANES, group_body, 0)

        sems = (sem_a, sem_b)
        pend = fire(0, sems[0])
        for q in range(NCHUNK):
            for cp in pend:
                cp.wait()
            if q + 1 < NCHUNK:
                pend = fire(q + 1, sems[(q + 1) % 2])
            compute(q)

        pltpu.sync_copy(outv, out_h.at[pl.ds(base, RPW)])

    us_rows = extract_kernel(users.astype(jnp.int32), ut)
    return combine_kernel(items, xij, us_rows, icat)


# speedup vs baseline: 1.0218x; 1.0218x over previous
import functools

import jax
import jax.numpy as jnp
from jax import lax
from jax.experimental import pallas as pl
from jax.experimental.pallas import tpu as pltpu
from jax.experimental.pallas import tpu_sc as plsc

LATENT = 64
XDIM = 16
UDIM = LATENT + XDIM
LANES = 16
WIDTH = 128
IDX_CHUNK = 128
SELCAP = 640
NRING = 4
SELCH = 2048


def kernel(users, items, xij, emb_user, emb_item, emb_item_xij1, emb_item_xij0):
    B = users.shape[0]
    NU = emb_user.shape[0]
    info = plsc.get_sparse_core_info()
    NC, NS = info.num_cores, info.num_subcores
    NW = NC * NS
    assert B % (NW * LANES) == 0
    RPW = B // NW
    NCHUNK = RPW // IDX_CHUNK
    NCOL = (NU + 127) // 128
    CPW = (NCOL + NW - 1) // NW
    BPAD = B + IDX_CHUNK
    NSEL = SELCAP // IDX_CHUNK

    ut = emb_user.T
    proj_i = jnp.eye(3 * LATENT // 2, WIDTH, dtype=jnp.float32)
    icat = jnp.concatenate(
        [jnp.exp(emb_item), emb_item_xij1, emb_item_xij0], axis=1) @ proj_i

    mesh = plsc.VectorSubcoreMesh(core_axis_name="c", subcore_axis_name="s")
    cparams = pltpu.CompilerParams(
        needs_layout_passes=False, use_tc_tiling_on_sc=True)

    @functools.partial(
        pl.kernel,
        out_type=jax.ShapeDtypeStruct((BPAD, WIDTH), jnp.float32),
        mesh=mesh,
        scratch_types=[
            pltpu.VMEM((2 * SELCH,), jnp.int32),
            pltpu.VMEM((NSEL, IDX_CHUNK), jnp.int32),
            pltpu.VMEM((SELCAP,), jnp.int32),
            pltpu.VMEM((NRING * UDIM, 128), jnp.float32),
            pltpu.VMEM((SELCAP, WIDTH), jnp.float32),
            pltpu.SemaphoreType.DMA,
            pltpu.SemaphoreType.DMA,
        ],
        compiler_params=cparams,
    )
    def extract_kernel(users_h, ut_h, out_h, uch, selr, selu, ring, orows,
                       sem_in, sem_out):
        wid = lax.axis_index("s") * NC + lax.axis_index("c")
        lo = wid * CPW
        lanes = lax.broadcasted_iota(jnp.int32, (LANES,), 0)

        def prefill(g, carry):
            idx = g * LANES + lanes
            plsc.store_scatter(selr, [lax.shift_right_logical(idx, 7),
                                      jnp.bitwise_and(idx, 127)],
                               jnp.full((LANES,), B, jnp.int32))
            selu[pl.ds(g * LANES, LANES)] = jnp.full((LANES,), 0x7FFFFFF,
                                                     jnp.int32)
            return carry
        lax.fori_loop(0, SELCAP // LANES, prefill, 0)

        def fire_chunk(ch):
            slot = lax.rem(ch, 2)

            @pl.when(ch < B // SELCH)
            def _():
                pltpu.async_copy(users_h.at[pl.ds(ch * SELCH, SELCH)],
                                 uch.at[pl.ds(slot * SELCH, SELCH)], sem_out)

        fire_chunk(0)

        def selchunk(ch, off0):
            slot = lax.rem(ch, 2)
            pltpu.make_async_copy(
                users_h.at[pl.ds(ch * SELCH, SELCH)],
                uch.at[pl.ds(slot * SELCH, SELCH)], sem_out).wait()
            fire_chunk(ch + 1)

            def select(g, off):
                u = uch[pl.ds(slot * SELCH + g * LANES, LANES)]
                tc = lax.shift_right_logical(u, 7) - lo
                m = (tc >= 0) & (tc < CPW)
                mi = m.astype(jnp.int32)
                pos = off + plsc.cumsum(mi) - 1
                m = m & (pos < SELCAP)
                plsc.store_scatter(selu, [pos], u, mask=m)
                plsc.store_scatter(selr, [lax.shift_right_logical(pos, 7),
                                          jnp.bitwise_and(pos, 127)],
                                   ch * SELCH + g * LANES + lanes, mask=m)
                return off + lax.reduce_sum(mi, axes=(0,))
            return lax.fori_loop(0, SELCH // LANES, select, off0)

        nsel = lax.fori_loop(0, B // SELCH, selchunk, jnp.int32(0))

        nsg = lax.div(nsel + LANES - 1, LANES)

        def fire(c):
            col = lo + c
            slot = lax.rem(c, NRING)

            @pl.when((col < NCOL) & (c < CPW))
            def _():
                pltpu.async_copy(
                    ut_h.at[:, pl.ds(col * 128, 128)],
                    ring.at[pl.ds(slot * UDIM, UDIM)], sem_in)

        for c in range(NRING - 1):
            fire(c)

        one = jnp.float32(1.0)

        def colbody(c, carry):
            col = lo + c
            slot = lax.rem(c, NRING)

            @pl.when(col < NCOL)
            def _wait():
                pltpu.make_async_copy(
                    ut_h.at[:, pl.ds(col * 128, 128)],
                    ring.at[pl.ds(slot * UDIM, UDIM)], sem_in).wait()

            fire(c + NRING - 1)

            @pl.when(col < NCOL)
            def _process():
                def scan_groups(g, carry2):
                    su = selu[pl.ds(g * LANES, LANES)]
                    m2 = lax.shift_right_logical(su, 7) == col

                    @pl.when(jnp.any(m2))
                    def _extract():
                        def member_loop(state):
                            mm, acc = state
                            lanev = plsc.all_reduce_ffs(mm)
                            uv = plsc.load_gather(selu, [g * LANES + lanev])
                            ucol = jnp.bitwise_and(uv, 127)
                            pos_v = g * LANES + lanev
                            for k in range(UDIM // LANES):
                                vals = plsc.load_gather(
                                    ring, [slot * UDIM + k * LANES + lanes,
                                           ucol])
                                sig = one / (one + jnp.exp(-vals))
                                plsc.store_scatter(
                                    orows, [pos_v, k * LANES + lanes], sig)
                            mm = mm & (lanes != lanev)
                            return (mm, acc)

                        lax.while_loop(lambda s: jnp.any(s[0]), member_loop,
                                       (m2, 0))
                    return carry2

                lax.fori_loop(0, nsg, scan_groups, 0)
            return carry

        lax.fori_loop(0, CPW, colbody, 0)

        sc_copies = []
        for j in range(NSEL):
            sc_copies.append(pltpu.async_copy(
                orows.at[pl.ds(j * IDX_CHUNK, IDX_CHUNK)],
                out_h.at[selr.at[j]], sem_out))
        for cp in sc_copies:
            cp.wait()

    @functools.partial(
        pl.kernel,
        out_type=jax.ShapeDtypeStruct((B,), jnp.float32),
        mesh=mesh,
        scratch_types=[
            pltpu.VMEM((RPW,), jnp.int32),
            pltpu.VMEM((RPW,), jnp.float32),
            pltpu.VMEM((RPW, WIDTH), jnp.float32),
            pltpu.VMEM((2 * IDX_CHUNK, WIDTH), jnp.float32),
            pltpu.VMEM((RPW,), jnp.float32),
            pltpu.SemaphoreType.DMA,
            pltpu.SemaphoreType.DMA,
        ],
        compiler_params=cparams,
    )
    def combine_kernel(items_h, xij_h, us_h, ic_h, out_h,
                       iidx, xv, ubuf, ibuf, outv, sem_a, sem_b):
        wid = lax.axis_index("s") * NC + lax.axis_index("c")
        base = wid * RPW

        bsl = pl.ds(base, RPW)
        c1 = pltpu.async_copy(items_h.at[bsl], iidx, sem_a)
        c2 = pltpu.async_copy(xij_h.at[bsl], xv, sem_a)
        c3 = pltpu.async_copy(us_h.at[bsl, :], ubuf, sem_b)
        c1.wait()
        c2.wait()
        c3.wait()

        lanes = lax.broadcasted_iota(jnp.int32, (LANES,), 0)
        one = jnp.float32(1.0)

        def fire(q, sem):
            sl = pl.ds((q % 2) * IDX_CHUNK, IDX_CHUNK)
            qsl = pl.ds(q * IDX_CHUNK, IDX_CHUNK)
            return (pltpu.async_copy(ic_h.at[iidx.at[qsl]], ibuf.at[sl], sem),)

        def compute(q):
            def group_body(g, carry):
                brows = (q % 2) * IDX_CHUNK + g * LANES + lanes
                off = q * IDX_CHUNK + g * LANES
                urows = off + lanes
                x = xv[pl.ds(off, LANES)]
                denom = jnp.zeros((LANES,), jnp.float32)
                numer = jnp.zeros((LANES,), jnp.float32)
                for d in range(LATENT):
                    dd = jnp.full((LANES,), d, jnp.int32)
                    e = plsc.load_gather(ibuf, [brows, dd])
                    s = plsc.load_gather(ubuf, [urows, dd])
                    denom = denom + e
                    numer = numer + s * e
                for d in range(XDIM):
                    d1 = jnp.full((LANES,), LATENT + d, jnp.int32)
                    d0 = jnp.full((LANES,), LATENT + XDIM + d, jnp.int32)
                    x1 = plsc.load_gather(ibuf, [brows, d1])
                    x0 = plsc.load_gather(ibuf, [brows, d0])
                    e = jnp.exp(x1 * x + x0 * (one - x))
                    s = plsc.load_gather(ubuf, [urows, d1])
                    denom = denom + e
                    numer = numer + s * e
                outv[pl.ds(off, LANES)] = numer / denom
                return carry

            lax.fori_loop(0, IDX_CHUNK // LANES, group_body, 0)

        sems = (sem_a, sem_b)
        pend = fire(0, sems[0])
        for q in range(NCHUNK):
            for cp in pend:
                cp.wait()
            if q + 1 < NCHUNK:
                pend = fire(q + 1, sems[(q + 1) % 2])
            compute(q)

        pltpu.sync_copy(outv, out_h.at[pl.ds(base, RPW)])

    us_rows = extract_kernel(users.astype(jnp.int32), ut)
    return combine_kernel(items, xij, us_rows, icat)


# speedup vs baseline: 1.2996x; 1.2718x over previous
import functools

import jax
import jax.numpy as jnp
from jax import lax
from jax.experimental import pallas as pl
from jax.experimental.pallas import tpu as pltpu
from jax.experimental.pallas import tpu_sc as plsc

LATENT = 64
XDIM = 16
UDIM = LATENT + XDIM
LANES = 16
WIDTH = 128
IDX_CHUNK = 128
SELCAP = 640
NRING = 3
SELCH = 2048


def _take16(vec, idxv):
    dnums = lax.GatherDimensionNumbers(
        offset_dims=(), collapsed_slice_dims=(0,), start_index_map=(0,))
    return lax.gather(vec, idxv[:, None], dnums, (1,),
                      mode=lax.GatherScatterMode.PROMISE_IN_BOUNDS)


def kernel(users, items, xij, emb_user, emb_item, emb_item_xij1, emb_item_xij0):
    B = users.shape[0]
    NU = emb_user.shape[0]
    info = plsc.get_sparse_core_info()
    NC, NS = info.num_cores, info.num_subcores
    NW = NC * NS
    assert B % (NW * LANES) == 0
    RPW = B // NW
    NCHUNK = RPW // IDX_CHUNK
    NCOL = (NU + 127) // 128
    CPW = (NCOL + NW - 1) // NW
    BPAD = B + IDX_CHUNK
    NSEL = SELCAP // IDX_CHUNK

    ut = emb_user.T
    proj_i = jnp.eye(3 * LATENT // 2, WIDTH, dtype=jnp.float32)
    icat = jnp.concatenate(
        [jnp.exp(emb_item), emb_item_xij1, emb_item_xij0], axis=1) @ proj_i

    mesh = plsc.VectorSubcoreMesh(core_axis_name="c", subcore_axis_name="s")
    cparams = pltpu.CompilerParams(
        needs_layout_passes=False, use_tc_tiling_on_sc=True)

    @functools.partial(
        pl.kernel,
        out_type=jax.ShapeDtypeStruct((BPAD, WIDTH), jnp.float32),
        mesh=mesh,
        scratch_types=[
            pltpu.VMEM((2 * SELCH,), jnp.int32),
            pltpu.VMEM((NSEL, IDX_CHUNK), jnp.int32),
            pltpu.VMEM((256 * 16,), jnp.int32),
            pltpu.VMEM((256 * 16,), jnp.int32),
            pltpu.VMEM((256,), jnp.int32),
            pltpu.VMEM((NRING * UDIM, 128), jnp.float32),
            pltpu.VMEM((SELCAP, WIDTH), jnp.float32),
            pltpu.SemaphoreType.DMA,
            pltpu.SemaphoreType.DMA,
        ],
        compiler_params=cparams,
    )
    def extract_kernel(users_h, ut_h, out_h, uch, selr, bktu, bktr, cnt,
                       ring, orows, sem_in, sem_out):
        wid = lax.axis_index("s") * NC + lax.axis_index("c")
        lo = wid * CPW
        lanes = lax.broadcasted_iota(jnp.int32, (LANES,), 0)

        def prefill(g, carry):
            idx = g * LANES + lanes
            plsc.store_scatter(selr, [lax.shift_right_logical(idx, 7),
                                      jnp.bitwise_and(idx, 127)],
                               jnp.full((LANES,), B, jnp.int32))
            return carry
        lax.fori_loop(0, SELCAP // LANES, prefill, 0)

        def zerocnt(g, carry):
            cnt[pl.ds(g * LANES, LANES)] = jnp.zeros((LANES,), jnp.int32)
            return carry
        lax.fori_loop(0, 256 // LANES, zerocnt, 0)

        def fire_chunk(ch):
            slot = lax.rem(ch, 2)

            @pl.when(ch < B // SELCH)
            def _():
                pltpu.async_copy(users_h.at[pl.ds(ch * SELCH, SELCH)],
                                 uch.at[pl.ds(slot * SELCH, SELCH)], sem_out)

        fire_chunk(0)

        def selchunk(ch, off0):
            slot = lax.rem(ch, 2)
            pltpu.make_async_copy(
                users_h.at[pl.ds(ch * SELCH, SELCH)],
                uch.at[pl.ds(slot * SELCH, SELCH)], sem_out).wait()
            fire_chunk(ch + 1)

            def select(g, carry):
                u = uch[pl.ds(slot * SELCH + g * LANES, LANES)]
                rows_v = ch * SELCH + g * LANES + lanes
                tc = lax.shift_right_logical(u, 7) - lo
                m = (tc >= 0) & (tc < CPW)

                def sel_member(state):
                    mm, c2 = state
                    lanev = plsc.all_reduce_ffs(mm)
                    tcv = _take16(tc, lanev)
                    uvv = _take16(u, lanev)
                    rvv = _take16(rows_v, lanev)
                    cv = plsc.load_gather(cnt, [tcv])
                    ok = (lanes == 0) & (cv < 16)
                    plsc.store_scatter(bktu, [tcv * 16 + cv], uvv, mask=ok)
                    plsc.store_scatter(bktr, [tcv * 16 + cv], rvv, mask=ok)
                    plsc.store_scatter(cnt, [tcv], cv + 1, mask=ok)
                    return (mm & (lanes != lanev), c2)

                lax.while_loop(lambda s: jnp.any(s[0]), sel_member, (m, 0))
                return carry
            return lax.fori_loop(0, SELCH // LANES, select, off0)

        lax.fori_loop(0, B // SELCH, selchunk, jnp.int32(0))

        def fire(c):
            col = lo + c
            slot = lax.rem(c, NRING)

            @pl.when((col < NCOL) & (c < CPW))
            def _():
                pltpu.async_copy(
                    ut_h.at[:, pl.ds(col * 128, 128)],
                    ring.at[pl.ds(slot * UDIM, UDIM)], sem_in)

        for c in range(NRING - 1):
            fire(c)

        one = jnp.float32(1.0)

        def colbody(c, mcount):
            col = lo + c
            slot = lax.rem(c, NRING)

            @pl.when(col < NCOL)
            def _wait():
                pltpu.make_async_copy(
                    ut_h.at[:, pl.ds(col * 128, 128)],
                    ring.at[pl.ds(slot * UDIM, UDIM)], sem_in).wait()

            fire(c + NRING - 1)

            cv = plsc.load_gather(cnt, [jnp.full((LANES,), 0, jnp.int32) + c])
            m0 = (lanes < cv) & jnp.full((LANES,), col < NCOL)

            def member_loop(state):
                mm, mc = state
                lanev = plsc.all_reduce_ffs(mm)
                uv = plsc.load_gather(bktu, [c * 16 + lanev])
                rv = plsc.load_gather(bktr, [c * 16 + lanev])
                ucol = jnp.bitwise_and(uv, 127)
                posv = jnp.zeros((LANES,), jnp.int32) + mc
                okw = posv < SELCAP
                for k in range(UDIM // LANES):
                    vals = plsc.load_gather(
                        ring, [slot * UDIM + k * LANES + lanes, ucol])
                    sig = one / (one + jnp.exp(-vals))
                    plsc.store_scatter(orows, [posv, k * LANES + lanes], sig,
                                       mask=okw)
                plsc.store_scatter(selr, [lax.shift_right_logical(posv, 7),
                                          jnp.bitwise_and(posv, 127)],
                                   rv, mask=okw & (lanes == 0))
                return (mm & (lanes != lanev), mc + 1)

            _, mcount = lax.while_loop(lambda s: jnp.any(s[0]), member_loop,
                                       (m0, mcount))
            return mcount

        lax.fori_loop(0, CPW, colbody, jnp.int32(0))

        sc_copies = []
        for j in range(NSEL):
            sc_copies.append(pltpu.async_copy(
                orows.at[pl.ds(j * IDX_CHUNK, IDX_CHUNK)],
                out_h.at[selr.at[j]], sem_out))
        for cp in sc_copies:
            cp.wait()

    @functools.partial(
        pl.kernel,
        out_type=jax.ShapeDtypeStruct((B,), jnp.float32),
        mesh=mesh,
        scratch_types=[
            pltpu.VMEM((RPW,), jnp.int32),
            pltpu.VMEM((RPW,), jnp.float32),
            pltpu.VMEM((RPW, WIDTH), jnp.float32),
            pltpu.VMEM((2 * IDX_CHUNK, WIDTH), jnp.float32),
            pltpu.VMEM((RPW,), jnp.float32),
            pltpu.SemaphoreType.DMA,
            pltpu.SemaphoreType.DMA,
        ],
        compiler_params=cparams,
    )
    def combine_kernel(items_h, xij_h, us_h, ic_h, out_h,
                       iidx, xv, ubuf, ibuf, outv, sem_a, sem_b):
        wid = lax.axis_index("s") * NC + lax.axis_index("c")
        base = wid * RPW

        bsl = pl.ds(base, RPW)
        c1 = pltpu.async_copy(items_h.at[bsl], iidx, sem_a)
        c2 = pltpu.async_copy(xij_h.at[bsl], xv, sem_a)
        c3 = pltpu.async_copy(us_h.at[bsl, :], ubuf, sem_b)
        c1.wait()
        c2.wait()
        c3.wait()

        lanes = lax.broadcasted_iota(jnp.int32, (LANES,), 0)
        one = jnp.float32(1.0)

        def fire(q, sem):
            sl = pl.ds((q % 2) * IDX_CHUNK, IDX_CHUNK)
            qsl = pl.ds(q * IDX_CHUNK, IDX_CHUNK)
            return (pltpu.async_copy(ic_h.at[iidx.at[qsl]], ibuf.at[sl], sem),)

        def compute(q):
            def group_body(g, carry):
                brows = (q % 2) * IDX_CHUNK + g * LANES + lanes
                off = q * IDX_CHUNK + g * LANES
                urows = off + lanes
                x = xv[pl.ds(off, LANES)]
                denom = jnp.zeros((LANES,), jnp.float32)
                numer = jnp.zeros((LANES,), jnp.float32)
                for d in range(LATENT):
                    dd = jnp.full((LANES,), d, jnp.int32)
                    e = plsc.load_gather(ibuf, [brows, dd])
                    s = plsc.load_gather(ubuf, [urows, dd])
                    denom = denom + e
                    numer = numer + s * e
                for d in range(XDIM):
                    d1 = jnp.full((LANES,), LATENT + d, jnp.int32)
                    d0 = jnp.full((LANES,), LATENT + XDIM + d, jnp.int32)
                    x1 = plsc.load_gather(ibuf, [brows, d1])
                    x0 = plsc.load_gather(ibuf, [brows, d0])
                    e = jnp.exp(x1 * x + x0 * (one - x))
                    s = plsc.load_gather(ubuf, [urows, d1])
                    denom = denom + e
                    numer = numer + s * e
                outv[pl.ds(off, LANES)] = numer / denom
                return carry

            lax.fori_loop(0, IDX_CHUNK // LANES, group_body, 0)

        sems = (sem_a, sem_b)
        pend = fire(0, sems[0])
        for q in range(NCHUNK):
            for cp in pend:
                cp.wait()
            if q + 1 < NCHUNK:
                pend = fire(q + 1, sems[(q + 1) % 2])
            compute(q)

        pltpu.sync_copy(outv, out_h.at[pl.ds(base, RPW)])

    us_rows = extract_kernel(users.astype(jnp.int32), ut)
    return combine_kernel(items, xij, us_rows, icat)


# speedup vs baseline: 1.3027x; 1.0024x over previous
import functools

import jax
import jax.numpy as jnp
from jax import lax
from jax.experimental import pallas as pl
from jax.experimental.pallas import tpu as pltpu
from jax.experimental.pallas import tpu_sc as plsc

LATENT = 64
XDIM = 16
UDIM = LATENT + XDIM
LANES = 16
WIDTH = 128
IDX_CHUNK = 128
SELCAP = 640
NRING = 3
SELCH = 2048


def _take16(vec, idxv):
    dnums = lax.GatherDimensionNumbers(
        offset_dims=(), collapsed_slice_dims=(0,), start_index_map=(0,))
    return lax.gather(vec, idxv[:, None], dnums, (1,),
                      mode=lax.GatherScatterMode.PROMISE_IN_BOUNDS)


def kernel(users, items, xij, emb_user, emb_item, emb_item_xij1, emb_item_xij0):
    B = users.shape[0]
    NU = emb_user.shape[0]
    info = plsc.get_sparse_core_info()
    NC, NS = info.num_cores, info.num_subcores
    NW = NC * NS
    assert B % (NW * LANES) == 0
    RPW = B // NW
    NCHUNK = RPW // IDX_CHUNK
    NCOL = (NU + 127) // 128
    CPW = (NCOL + NW - 1) // NW
    BPAD = B + IDX_CHUNK
    NSEL = SELCAP // IDX_CHUNK

    ut = emb_user.T
    proj_i = jnp.eye(3 * LATENT // 2, WIDTH, dtype=jnp.float32)
    icat = jnp.concatenate(
        [jnp.exp(emb_item), emb_item_xij1, emb_item_xij0], axis=1) @ proj_i

    mesh = plsc.VectorSubcoreMesh(core_axis_name="c", subcore_axis_name="s")
    cparams = pltpu.CompilerParams(
        needs_layout_passes=False, use_tc_tiling_on_sc=True)

    @functools.partial(
        pl.kernel,
        out_type=jax.ShapeDtypeStruct((BPAD, WIDTH), jnp.float32),
        mesh=mesh,
        scratch_types=[
            pltpu.VMEM((2 * SELCH,), jnp.int32),
            pltpu.VMEM((NSEL, IDX_CHUNK), jnp.int32),
            pltpu.VMEM((256 * 16,), jnp.int32),
            pltpu.VMEM((256 * 16,), jnp.int32),
            pltpu.VMEM((256,), jnp.int32),
            pltpu.VMEM((NRING * UDIM, 128), jnp.float32),
            pltpu.VMEM((SELCAP, WIDTH), jnp.float32),
            pltpu.SemaphoreType.DMA,
            pltpu.SemaphoreType.DMA,
        ],
        compiler_params=cparams,
    )
    def extract_kernel(users_h, ut_h, out_h, uch, selr, bktu, bktr, cnt,
                       ring, orows, sem_in, sem_out):
        wid = lax.axis_index("s") * NC + lax.axis_index("c")
        lo = wid * CPW
        lanes = lax.broadcasted_iota(jnp.int32, (LANES,), 0)

        def prefill(g, carry):
            idx = g * LANES + lanes
            plsc.store_scatter(selr, [lax.shift_right_logical(idx, 7),
                                      jnp.bitwise_and(idx, 127)],
                               jnp.full((LANES,), B, jnp.int32))
            return carry
        lax.fori_loop(0, SELCAP // LANES, prefill, 0)

        def zerocnt(g, carry):
            cnt[pl.ds(g * LANES, LANES)] = jnp.zeros((LANES,), jnp.int32)
            return carry
        lax.fori_loop(0, 256 // LANES, zerocnt, 0)

        def fire_chunk(ch):
            slot = lax.rem(ch, 2)

            @pl.when(ch < B // SELCH)
            def _():
                pltpu.async_copy(users_h.at[pl.ds(ch * SELCH, SELCH)],
                                 uch.at[pl.ds(slot * SELCH, SELCH)], sem_out)

        fire_chunk(0)

        def selchunk(ch, off0):
            slot = lax.rem(ch, 2)
            pltpu.make_async_copy(
                users_h.at[pl.ds(ch * SELCH, SELCH)],
                uch.at[pl.ds(slot * SELCH, SELCH)], sem_out).wait()
            fire_chunk(ch + 1)

            def select(g, carry):
                u = uch[pl.ds(slot * SELCH + g * LANES, LANES)]
                rows_v = ch * SELCH + g * LANES + lanes
                tc = lax.shift_right_logical(u, 7) - lo
                m = (tc >= 0) & (tc < CPW)

                def sel_member(state):
                    mm, c2 = state
                    lanev = plsc.all_reduce_ffs(mm)
                    tcv = _take16(tc, lanev)
                    uvv = _take16(u, lanev)
                    rvv = _take16(rows_v, lanev)
                    cv = plsc.load_gather(cnt, [tcv])
                    ok = (lanes == 0) & (cv < 16)
                    plsc.store_scatter(bktu, [tcv * 16 + cv], uvv, mask=ok)
                    plsc.store_scatter(bktr, [tcv * 16 + cv], rvv, mask=ok)
                    plsc.store_scatter(cnt, [tcv], cv + 1, mask=ok)
                    return (mm & (lanes != lanev), c2)

                lax.while_loop(lambda s: jnp.any(s[0]), sel_member, (m, 0))
                return carry
            return lax.fori_loop(0, SELCH // LANES, select, off0)

        lax.fori_loop(0, B // SELCH, selchunk, jnp.int32(0))

        def fire(c):
            col = lo + c
            slot = lax.rem(c, NRING)

            @pl.when((col < NCOL) & (c < CPW))
            def _():
                for tr in range(UDIM // 8):
                    pltpu.async_copy(
                        ut_h.at[pl.ds(tr * 8, 8), pl.ds(col * 128, 128)],
                        ring.at[pl.ds(slot * UDIM + tr * 8, 8)], sem_in)

        for c in range(NRING - 1):
            fire(c)

        one = jnp.float32(1.0)

        def colbody(c, mcount):
            col = lo + c
            slot = lax.rem(c, NRING)

            @pl.when(col < NCOL)
            def _wait():
                pltpu.make_async_copy(
                    ut_h.at[:, pl.ds(col * 128, 128)],
                    ring.at[pl.ds(slot * UDIM, UDIM)], sem_in).wait()

            fire(c + NRING - 1)

            cv = plsc.load_gather(cnt, [jnp.full((LANES,), 0, jnp.int32) + c])
            m0 = (lanes < cv) & jnp.full((LANES,), col < NCOL)

            def member_loop(state):
                mm, mc = state
                lanev = plsc.all_reduce_ffs(mm)
                uv = plsc.load_gather(bktu, [c * 16 + lanev])
                rv = plsc.load_gather(bktr, [c * 16 + lanev])
                ucol = jnp.bitwise_and(uv, 127)
                posv = jnp.zeros((LANES,), jnp.int32) + mc
                okw = posv < SELCAP
                for k in range(UDIM // LANES):
                    vals = plsc.load_gather(
                        ring, [slot * UDIM + k * LANES + lanes, ucol])
                    sig = one / (one + jnp.exp(-vals))
                    plsc.store_scatter(orows, [posv, k * LANES + lanes], sig,
                                       mask=okw)
                plsc.store_scatter(selr, [lax.shift_right_logical(posv, 7),
                                          jnp.bitwise_and(posv, 127)],
                                   rv, mask=okw & (lanes == 0))
                return (mm & (lanes != lanev), mc + 1)

            _, mcount = lax.while_loop(lambda s: jnp.any(s[0]), member_loop,
                                       (m0, mcount))
            return mcount

        lax.fori_loop(0, CPW, colbody, jnp.int32(0))

        sc_copies = []
        for j in range(NSEL):
            sc_copies.append(pltpu.async_copy(
                orows.at[pl.ds(j * IDX_CHUNK, IDX_CHUNK)],
                out_h.at[selr.at[j]], sem_out))
        for cp in sc_copies:
            cp.wait()

    @functools.partial(
        pl.kernel,
        out_type=jax.ShapeDtypeStruct((B,), jnp.float32),
        mesh=mesh,
        scratch_types=[
            pltpu.VMEM((RPW,), jnp.int32),
            pltpu.VMEM((RPW,), jnp.float32),
            pltpu.VMEM((RPW, WIDTH), jnp.float32),
            pltpu.VMEM((2 * IDX_CHUNK, WIDTH), jnp.float32),
            pltpu.VMEM((RPW,), jnp.float32),
            pltpu.SemaphoreType.DMA,
            pltpu.SemaphoreType.DMA,
        ],
        compiler_params=cparams,
    )
    def combine_kernel(items_h, xij_h, us_h, ic_h, out_h,
                       iidx, xv, ubuf, ibuf, outv, sem_a, sem_b):
        wid = lax.axis_index("s") * NC + lax.axis_index("c")
        base = wid * RPW

        bsl = pl.ds(base, RPW)
        c1 = pltpu.async_copy(items_h.at[bsl], iidx, sem_a)
        c2 = pltpu.async_copy(xij_h.at[bsl], xv, sem_a)
        c3 = pltpu.async_copy(us_h.at[bsl, :], ubuf, sem_b)
        c1.wait()
        c2.wait()
        c3.wait()

        lanes = lax.broadcasted_iota(jnp.int32, (LANES,), 0)
        one = jnp.float32(1.0)

        def fire(q, sem):
            sl = pl.ds((q % 2) * IDX_CHUNK, IDX_CHUNK)
            qsl = pl.ds(q * IDX_CHUNK, IDX_CHUNK)
            return (pltpu.async_copy(ic_h.at[iidx.at[qsl]], ibuf.at[sl], sem),)

        def compute(q):
            def group_body(g, carry):
                brows = (q % 2) * IDX_CHUNK + g * LANES + lanes
                off = q * IDX_CHUNK + g * LANES
                urows = off + lanes
                x = xv[pl.ds(off, LANES)]
                denom = jnp.zeros((LANES,), jnp.float32)
                numer = jnp.zeros((LANES,), jnp.float32)
                for d in range(LATENT):
                    dd = jnp.full((LANES,), d, jnp.int32)
                    e = plsc.load_gather(ibuf, [brows, dd])
                    s = plsc.load_gather(ubuf, [urows, dd])
                    denom = denom + e
                    numer = numer + s * e
                for d in range(XDIM):
                    d1 = jnp.full((LANES,), LATENT + d, jnp.int32)
                    d0 = jnp.full((LANES,), LATENT + XDIM + d, jnp.int32)
                    x1 = plsc.load_gather(ibuf, [brows, d1])
                    x0 = plsc.load_gather(ibuf, [brows, d0])
                    e = jnp.exp(x1 * x + x0 * (one - x))
                    s = plsc.load_gather(ubuf, [urows, d1])
                    denom = denom + e
                    numer = numer + s * e
                outv[pl.ds(off, LANES)] = numer / denom
                return carry

            lax.fori_loop(0, IDX_CHUNK // LANES, group_body, 0)

        sems = (sem_a, sem_b)
        pend = fire(0, sems[0])
        for q in range(NCHUNK):
            for cp in pend:
                cp.wait()
            if q + 1 < NCHUNK:
                pend = fire(q + 1, sems[(q + 1) % 2])
            compute(q)

        pltpu.sync_copy(outv, out_h.at[pl.ds(base, RPW)])

    us_rows = extract_kernel(users.astype(jnp.int32), ut)
    return combine_kernel(items, xij, us_rows, icat)


# speedup vs baseline: 1.5480x; 1.1883x over previous
import functools

import jax
import jax.numpy as jnp
from jax import lax
from jax.experimental import pallas as pl
from jax.experimental.pallas import tpu as pltpu
from jax.experimental.pallas import tpu_sc as plsc

LATENT = 64
XDIM = 16
UDIM = LATENT + XDIM
LANES = 16
WIDTH = 128
IDX_CHUNK = 128


def kernel(users, items, xij, emb_user, emb_item, emb_item_xij1, emb_item_xij0):
    B = users.shape[0]
    info = plsc.get_sparse_core_info()
    NC, NS = info.num_cores, info.num_subcores
    NW = NC * NS
    assert B % (NW * LANES) == 0
    RPW = B // NW
    NCHUNK = RPW // IDX_CHUNK

    proj_u = jnp.eye(UDIM, WIDTH, dtype=jnp.float32)
    us_p = jax.nn.sigmoid(emb_user) @ proj_u
    proj_i = jnp.eye(3 * LATENT // 2, WIDTH, dtype=jnp.float32)
    icat = jnp.concatenate(
        [jnp.exp(emb_item), emb_item_xij1, emb_item_xij0], axis=1) @ proj_i

    mesh = plsc.VectorSubcoreMesh(core_axis_name="c", subcore_axis_name="s")

    @functools.partial(
        pl.kernel,
        out_type=jax.ShapeDtypeStruct((B,), jnp.float32),
        mesh=mesh,
        scratch_types=[
            pltpu.VMEM((RPW,), jnp.int32),
            pltpu.VMEM((RPW,), jnp.int32),
            pltpu.VMEM((RPW,), jnp.float32),
            pltpu.VMEM((2 * IDX_CHUNK, WIDTH), jnp.float32),
            pltpu.VMEM((2 * IDX_CHUNK, WIDTH), jnp.float32),
            pltpu.VMEM((RPW,), jnp.float32),
            pltpu.SemaphoreType.DMA,
            pltpu.SemaphoreType.DMA,
        ],
        compiler_params=pltpu.CompilerParams(
            needs_layout_passes=False, use_tc_tiling_on_sc=True),
    )
    def sc_kernel(users_h, items_h, xij_h, eu_h, ic_h, out_h,
                  uidx, iidx, xv, ubuf, ibuf, outv, sem_a, sem_b):
        wid = lax.axis_index("s") * NC + lax.axis_index("c")
        base = wid * RPW

        bsl = pl.ds(base, RPW)
        c1 = pltpu.async_copy(users_h.at[bsl], uidx, sem_a)
        c2 = pltpu.async_copy(items_h.at[bsl], iidx, sem_a)
        c3 = pltpu.async_copy(xij_h.at[bsl], xv, sem_a)
        c1.wait()
        c2.wait()
        c3.wait()

        lanes = lax.broadcasted_iota(jnp.int32, (LANES,), 0)
        one = jnp.float32(1.0)

        def fire(q, sem):
            sl = pl.ds((q % 2) * IDX_CHUNK, IDX_CHUNK)
            qsl = pl.ds(q * IDX_CHUNK, IDX_CHUNK)
            cu = pltpu.async_copy(eu_h.at[uidx.at[qsl]], ubuf.at[sl], sem)
            ci = pltpu.async_copy(ic_h.at[iidx.at[qsl]], ibuf.at[sl], sem)
            return (cu, ci)

        def compute(q):
            def group_body(g, carry):
                rows = (q % 2) * IDX_CHUNK + g * LANES + lanes
                off = q * IDX_CHUNK + g * LANES
                x = xv[pl.ds(off, LANES)]
                denom = jnp.zeros((LANES,), jnp.float32)
                numer = jnp.zeros((LANES,), jnp.float32)
                for d in range(LATENT):
                    dd = jnp.full((LANES,), d, jnp.int32)
                    e = plsc.load_gather(ibuf, [rows, dd])
                    s = plsc.load_gather(ubuf, [rows, dd])
                    denom = denom + e
                    numer = numer + s * e
                for d in range(XDIM):
                    d1 = jnp.full((LANES,), LATENT + d, jnp.int32)
                    d0 = jnp.full((LANES,), LATENT + XDIM + d, jnp.int32)
                    x1 = plsc.load_gather(ibuf, [rows, d1])
                    x0 = plsc.load_gather(ibuf, [rows, d0])
                    e = jnp.exp(x1 * x + x0 * (one - x))
                    s = plsc.load_gather(ubuf, [rows, d1])
                    denom = denom + e
                    numer = numer + s * e
                outv[pl.ds(off, LANES)] = numer / denom
                return carry

            lax.fori_loop(0, IDX_CHUNK // LANES, group_body, 0)

        sems = (sem_a, sem_b)
        pend = fire(0, sems[0])
        for q in range(NCHUNK):
            for cp in pend:
                cp.wait()
            if q + 1 < NCHUNK:
                pend = fire(q + 1, sems[(q + 1) % 2])
            compute(q)

        pltpu.sync_copy(outv, out_h.at[pl.ds(base, RPW)])

    return sc_kernel(users.astype(jnp.int32), items, xij, us_p, icat)
